# baseline (speedup 1.0000x reference)
import functools

import jax
import jax.numpy as jnp
from jax import lax
from jax.experimental import pallas as pl
from jax.experimental.pallas import tpu as pltpu
from jax.experimental.pallas import tpu_sc as plsc

_NUM_CORES = 2
_NUM_SUBCORES = 16
_NUM_WORKERS = _NUM_CORES * _NUM_SUBCORES
_BLK = 128
_LANES = 16

WORD_DIM = 128
POS_SIZE = 16
POS_DIM = 2 * POS_SIZE
OUT_DIM = WORD_DIM + POS_DIM


@functools.partial(jax.jit, static_argnames=("b", "l"))
def _embed(wflat, p1flat, p2flat, word_table, p1v, p2v, b, l):
    n_blocks = (b // _BLK) * l
    per_w = n_blocks // _NUM_WORKERS
    assert per_w % 2 == 0 and per_w >= 4
    bt_per_l = b // _BLK
    pos_words = word_table.shape[0] and p1v.shape[0]
    mesh = plsc.VectorSubcoreMesh(
        core_axis_name="c", subcore_axis_name="s", num_cores=_NUM_CORES
    )
    wt_bytes = _BLK * WORD_DIM * 4
    pb_bytes = _BLK * POS_DIM * 4

    @functools.partial(
        pl.kernel,
        out_type=jax.ShapeDtypeStruct((l, OUT_DIM, b), jnp.float32),
        mesh=mesh,
        scratch_types=[
            pltpu.VMEM((512 * (POS_SIZE + 1),), jnp.float32),
            pltpu.VMEM((512 * (POS_SIZE + 1),), jnp.float32),
            pltpu.VMEM((_BLK,), jnp.int32),
            pltpu.VMEM((_BLK,), jnp.int32),
            pltpu.VMEM((_BLK,), jnp.int32),
            pltpu.VMEM((_BLK, WORD_DIM), jnp.float32),
            pltpu.VMEM((WORD_DIM, _BLK), jnp.float32),
            pltpu.VMEM((POS_DIM, _BLK), jnp.float32),
            pltpu.VMEM((_BLK,), jnp.int32),
            pltpu.VMEM((_BLK,), jnp.int32),
            pltpu.VMEM((_BLK,), jnp.int32),
            pltpu.VMEM((_BLK, WORD_DIM), jnp.float32),
            pltpu.VMEM((WORD_DIM, _BLK), jnp.float32),
            pltpu.VMEM((POS_DIM, _BLK), jnp.float32),
            pltpu.SemaphoreType.DMA,
            pltpu.SemaphoreType.DMA,
            pltpu.SemaphoreType.DMA,
            pltpu.SemaphoreType.DMA,
        ],
        compiler_params=pltpu.CompilerParams(
            use_tc_tiling_on_sc=True,
            needs_layout_passes=False,
            disable_bounds_checks=True,
        ),
    )
    def emb_kernel(
        wflat_hbm,
        p1flat_hbm,
        p2flat_hbm,
        wt_hbm,
        p1v_hbm,
        p2v_hbm,
        z_hbm,
        p1tab,
        p2tab,
        *bufs_and_sems,
    ):
        buf_a = bufs_and_sems[0:6]
        buf_b = bufs_and_sems[6:12]
        gsem_a, gsem_b, wsem_a, wsem_b = bufs_and_sems[12:16]
        wid = lax.axis_index("s") * _NUM_CORES + lax.axis_index("c")
        blk0 = wid * per_w

        pltpu.sync_copy(p1v_hbm, p1tab)
        pltpu.sync_copy(p2v_hbm, p2tab)

        iota = lax.iota(jnp.int32, _LANES)
        tvecs = [iota + g * _LANES for g in range(_BLK // _LANES)]

        def issue(blk, buf, gsem):
            widx, i1, i2, rows, _, _ = buf
            off = blk * _BLK
            pltpu.sync_copy(wflat_hbm.at[pl.ds(off, _BLK)], widx)
            pltpu.sync_copy(p1flat_hbm.at[pl.ds(off, _BLK)], i1)
            pltpu.sync_copy(p2flat_hbm.at[pl.ds(off, _BLK)], i2)
            pltpu.async_copy(wt_hbm.at[widx], rows, gsem)

        def z_slices(blk):
            li = blk // bt_per_l
            bt = blk % bt_per_l
            zw = z_hbm.at[li, pl.ds(0, WORD_DIM), pl.ds(bt * _BLK, _BLK)]
            zp = z_hbm.at[li, pl.ds(WORD_DIM, POS_DIM), pl.ds(bt * _BLK, _BLK)]
            return zw, zp

        def process(blk, buf, gsem, wsem, first=False):
            widx, i1, i2, rows, trows, pblk = buf
            zw, zp = z_slices(blk)
            if not first:
                pltpu.make_async_copy(trows, zw, wsem).wait()
                pltpu.make_async_copy(pblk, zp, wsem).wait()

            @plsc.parallel_loop(0, _BLK // _LANES, unroll=2)
            def _pg(g):
                sl = pl.ds(g * _LANES, _LANES)
                b1 = i1[sl] * (POS_SIZE + 1)
                b2 = i2[sl] * (POS_SIZE + 1)
                for f in range(POS_SIZE):
                    pblk[f, sl] = plsc.load_gather(p1tab, [b1 + f])
                    pblk[POS_SIZE + f, sl] = plsc.load_gather(p2tab, [b2 + f])

            pltpu.make_async_copy(wt_hbm.at[widx], rows, gsem).wait()
            @plsc.parallel_loop(0, WORD_DIM, unroll=8)
            def _f(i):
                gf = i >> 4
                d = i & 15
                col = (gf * _LANES) + ((d + iota) & 15)
                for g in range(_BLK // _LANES):
                    v = plsc.load_gather(rows, [tvecs[g], col])
                    plsc.store_scatter(trows, [col, tvecs[g]], v)

            pltpu.async_copy(trows, zw, wsem)
            pltpu.async_copy(pblk, zp, wsem)

        issue(blk0, buf_a, gsem_a)
        issue(blk0 + 1, buf_b, gsem_b)
        process(blk0, buf_a, gsem_a, wsem_a, first=True)
        issue(blk0 + 2, buf_a, gsem_a)
        process(blk0 + 1, buf_b, gsem_b, wsem_b, first=True)
        issue(blk0 + 3, buf_b, gsem_b)

        @pl.loop(1, per_w // 2 - 1)
        def _pair(k):
            s = blk0 + 2 * k
            process(s, buf_a, gsem_a, wsem_a)
            issue(s + 2, buf_a, gsem_a)
            process(s + 1, buf_b, gsem_b, wsem_b)
            issue(s + 3, buf_b, gsem_b)

        process(blk0 + per_w - 2, buf_a, gsem_a, wsem_a)
        process(blk0 + per_w - 1, buf_b, gsem_b, wsem_b)
        zw, zp = z_slices(blk0 + per_w - 2)
        pltpu.make_async_copy(buf_a[4], zw, wsem_a).wait()
        pltpu.make_async_copy(buf_a[5], zp, wsem_a).wait()
        zw, zp = z_slices(blk0 + per_w - 1)
        pltpu.make_async_copy(buf_b[4], zw, wsem_b).wait()
        pltpu.make_async_copy(buf_b[5], zp, wsem_b).wait()

    return emb_kernel(wflat, p1flat, p2flat, word_table, p1v, p2v)


def kernel(word, pos1, pos2, word_table, pos1_table, pos2_table):
    b, l = word.shape
    assert b % _BLK == 0 and (b // _BLK) * l % (2 * _NUM_WORKERS) == 0
    wflat = jnp.transpose(word).reshape(-1).astype(jnp.int32)
    p1flat = jnp.transpose(pos1).reshape(-1).astype(jnp.int32)
    p2flat = jnp.transpose(pos2).reshape(-1).astype(jnp.int32)
    p1pad = jnp.pad(pos1_table, ((0, 0), (0, 1))).reshape(-1)
    p2pad = jnp.pad(pos2_table, ((0, 0), (0, 1))).reshape(-1)
    z = _embed(wflat, p1flat, p2flat, word_table, p1pad, p2pad, b, l)
    return jnp.transpose(z, (2, 0, 1))


# speedup vs baseline: 1.0789x; 1.0789x over previous
import functools

import jax
import jax.numpy as jnp
from jax import lax
from jax.experimental import pallas as pl
from jax.experimental.pallas import tpu as pltpu
from jax.experimental.pallas import tpu_sc as plsc

_NUM_CORES = 2
_NUM_SUBCORES = 16
_NUM_WORKERS = _NUM_CORES * _NUM_SUBCORES
_BLK = 128
_LANES = 16

WORD_DIM = 128
POS_SIZE = 16
POS_DIM = 2 * POS_SIZE
OUT_DIM = WORD_DIM + POS_DIM


@functools.partial(jax.jit, static_argnames=("b", "l"))
def _embed(wflat, p1flat, p2flat, word_table, p1v, p2v, b, l):
    n_blocks = (b // _BLK) * l
    per_w = n_blocks // _NUM_WORKERS
    assert per_w % 2 == 0 and per_w >= 4
    bt_per_l = b // _BLK
    mesh = plsc.VectorSubcoreMesh(
        core_axis_name="c", subcore_axis_name="s", num_cores=_NUM_CORES
    )
    wt_bytes = _BLK * WORD_DIM * 4
    pb_bytes = _BLK * POS_DIM * 4

    @functools.partial(
        pl.kernel,
        out_type=jax.ShapeDtypeStruct((l, OUT_DIM, b), jnp.float32),
        mesh=mesh,
        scratch_types=[
            pltpu.VMEM((512 * (POS_SIZE + 1),), jnp.float32),
            pltpu.VMEM((512 * (POS_SIZE + 1),), jnp.float32),
            pltpu.VMEM((_BLK,), jnp.int32),
            pltpu.VMEM((_BLK,), jnp.int32),
            pltpu.VMEM((_BLK,), jnp.int32),
            pltpu.VMEM((_BLK, WORD_DIM), jnp.float32),
            pltpu.VMEM((WORD_DIM, _BLK), jnp.float32),
            pltpu.VMEM((POS_DIM, _BLK), jnp.float32),
            pltpu.VMEM((_BLK,), jnp.int32),
            pltpu.VMEM((_BLK,), jnp.int32),
            pltpu.VMEM((_BLK,), jnp.int32),
            pltpu.VMEM((_BLK, WORD_DIM), jnp.float32),
            pltpu.VMEM((WORD_DIM, _BLK), jnp.float32),
            pltpu.VMEM((POS_DIM, _BLK), jnp.float32),
            pltpu.SemaphoreType.DMA,
            pltpu.SemaphoreType.DMA,
            pltpu.SemaphoreType.DMA,
            pltpu.SemaphoreType.DMA,
        ],
        compiler_params=pltpu.CompilerParams(
            use_tc_tiling_on_sc=True,
            needs_layout_passes=False,
            disable_bounds_checks=True,
        ),
    )
    def emb_kernel(
        wflat_hbm,
        p1flat_hbm,
        p2flat_hbm,
        wt_hbm,
        p1v_hbm,
        p2v_hbm,
        z_hbm,
        p1tab,
        p2tab,
        *bufs_and_sems,
    ):
        buf_a = bufs_and_sems[0:6]
        buf_b = bufs_and_sems[6:12]
        gsem_a, gsem_b, wsem_a, wsem_b = bufs_and_sems[12:16]
        wid = lax.axis_index("s") * _NUM_CORES + lax.axis_index("c")
        blk0 = wid * per_w

        pltpu.sync_copy(p1v_hbm, p1tab)
        pltpu.sync_copy(p2v_hbm, p2tab)

        iota = lax.iota(jnp.int32, _LANES)
        tvecs = [iota + g * _LANES for g in range(_BLK // _LANES)]

        def issue(blk, buf, gsem):
            widx, i1, i2, rows, _, _ = buf
            off = blk * _BLK
            pltpu.sync_copy(wflat_hbm.at[pl.ds(off, _BLK)], widx)
            pltpu.sync_copy(p1flat_hbm.at[pl.ds(off, _BLK)], i1)
            pltpu.sync_copy(p2flat_hbm.at[pl.ds(off, _BLK)], i2)
            pltpu.async_copy(wt_hbm.at[widx], rows, gsem)

        def z_slices(blk):
            li = blk // bt_per_l
            bt = blk % bt_per_l
            zw = z_hbm.at[li, pl.ds(0, WORD_DIM), pl.ds(bt * _BLK, _BLK)]
            zp = z_hbm.at[li, pl.ds(WORD_DIM, POS_DIM), pl.ds(bt * _BLK, _BLK)]
            return zw, zp

        def process(blk, buf, gsem, wsem, first=False):
            widx, i1, i2, rows, trows, pblk = buf
            zw, zp = z_slices(blk)
            if not first:
                pltpu.make_async_copy(trows, zw, wsem).wait()
                pltpu.make_async_copy(pblk, zp, wsem).wait()

            @plsc.parallel_loop(0, _BLK // _LANES, unroll=2)
            def _pg(g):
                sl = pl.ds(g * _LANES, _LANES)
                b1 = i1[sl] * (POS_SIZE + 1)
                b2 = i2[sl] * (POS_SIZE + 1)
                for f in range(POS_SIZE):
                    pblk[f, sl] = plsc.load_gather(p1tab, [b1 + f])
                    pblk[POS_SIZE + f, sl] = plsc.load_gather(p2tab, [b2 + f])

            pltpu.make_async_copy(wt_hbm.at[widx], rows, gsem).wait()
            @plsc.parallel_loop(0, WORD_DIM, unroll=4)
            def _f(i):
                gf = i >> 4
                d = i & 15
                col = (gf * _LANES) + ((d + iota) & 15)
                for g in range(_BLK // _LANES):
                    v = plsc.load_gather(rows, [tvecs[g], col])
                    plsc.store_scatter(trows, [col, tvecs[g]], v)

            pltpu.async_copy(trows, zw, wsem)
            pltpu.async_copy(pblk, zp, wsem)

        issue(blk0, buf_a, gsem_a)
        issue(blk0 + 1, buf_b, gsem_b)
        process(blk0, buf_a, gsem_a, wsem_a, first=True)
        issue(blk0 + 2, buf_a, gsem_a)
        process(blk0 + 1, buf_b, gsem_b, wsem_b, first=True)
        issue(blk0 + 3, buf_b, gsem_b)

        @pl.loop(1, per_w // 2 - 1)
        def _pair(k):
            s = blk0 + 2 * k
            process(s, buf_a, gsem_a, wsem_a)
            issue(s + 2, buf_a, gsem_a)
            process(s + 1, buf_b, gsem_b, wsem_b)
            issue(s + 3, buf_b, gsem_b)

        process(blk0 + per_w - 2, buf_a, gsem_a, wsem_a)
        process(blk0 + per_w - 1, buf_b, gsem_b, wsem_b)
        zw, zp = z_slices(blk0 + per_w - 2)
        pltpu.make_async_copy(buf_a[4], zw, wsem_a).wait()
        pltpu.make_async_copy(buf_a[5], zp, wsem_a).wait()
        zw, zp = z_slices(blk0 + per_w - 1)
        pltpu.make_async_copy(buf_b[4], zw, wsem_b).wait()
        pltpu.make_async_copy(buf_b[5], zp, wsem_b).wait()

    return emb_kernel(wflat, p1flat, p2flat, word_table, p1v, p2v)


def kernel(word, pos1, pos2, word_table, pos1_table, pos2_table):
    b, l = word.shape
    assert b % _BLK == 0 and (b // _BLK) * l % (2 * _NUM_WORKERS) == 0
    wflat = jnp.transpose(word).reshape(-1).astype(jnp.int32)
    p1flat = jnp.transpose(pos1).reshape(-1).astype(jnp.int32)
    p2flat = jnp.transpose(pos2).reshape(-1).astype(jnp.int32)
    p1pad = jnp.pad(pos1_table, ((0, 0), (0, 1))).reshape(-1)
    p2pad = jnp.pad(pos2_table, ((0, 0), (0, 1))).reshape(-1)
    z = _embed(wflat, p1flat, p2flat, word_table, p1pad, p2pad, b, l)
    return jnp.transpose(z, (2, 0, 1))


# speedup vs baseline: 1.5631x; 1.4488x over previous
import functools

import jax
import jax.numpy as jnp
from jax import lax
from jax.experimental import pallas as pl
from jax.experimental.pallas import tpu as pltpu
from jax.experimental.pallas import tpu_sc as plsc

_NUM_CORES = 2
_NUM_SUBCORES = 16
_NUM_WORKERS = _NUM_CORES * _NUM_SUBCORES
_BLK = 128
_LANES = 16

WORD_DIM = 128
POS_SIZE = 16
POS_DIM = 2 * POS_SIZE
OUT_DIM = WORD_DIM + POS_DIM


@functools.partial(jax.jit, static_argnames=("b", "l"))
def _embed(wflat, p1flat, p2flat, word_table, p1v, p2v, b, l):
    n_blocks = (b // _BLK) * l
    per_w = n_blocks // _NUM_WORKERS
    assert per_w % 2 == 0 and per_w >= 4
    bt_per_l = b // _BLK
    mesh = plsc.VectorSubcoreMesh(
        core_axis_name="c", subcore_axis_name="s", num_cores=_NUM_CORES
    )
    wt_bytes = _BLK * WORD_DIM * 4
    pb_bytes = _BLK * POS_DIM * 4

    @functools.partial(
        pl.kernel,
        out_type=jax.ShapeDtypeStruct((l, OUT_DIM, b), jnp.float32),
        mesh=mesh,
        scratch_types=[
            pltpu.VMEM((512 * (POS_SIZE + 1),), jnp.float32),
            pltpu.VMEM((512 * (POS_SIZE + 1),), jnp.float32),
            pltpu.VMEM((per_w * _BLK,), jnp.int32),
            pltpu.VMEM((per_w * _BLK,), jnp.int32),
            pltpu.VMEM((per_w * _BLK,), jnp.int32),
            pltpu.VMEM((_BLK, WORD_DIM), jnp.float32),
            pltpu.VMEM((WORD_DIM, _BLK), jnp.float32),
            pltpu.VMEM((POS_DIM, _BLK), jnp.float32),
            pltpu.VMEM((_BLK, WORD_DIM), jnp.float32),
            pltpu.VMEM((WORD_DIM, _BLK), jnp.float32),
            pltpu.VMEM((POS_DIM, _BLK), jnp.float32),
            pltpu.SemaphoreType.DMA,
            pltpu.SemaphoreType.DMA,
            pltpu.SemaphoreType.DMA,
            pltpu.SemaphoreType.DMA,
        ],
        compiler_params=pltpu.CompilerParams(
            use_tc_tiling_on_sc=True,
            needs_layout_passes=False,
            disable_bounds_checks=True,
        ),
    )
    def emb_kernel(
        wflat_hbm,
        p1flat_hbm,
        p2flat_hbm,
        wt_hbm,
        p1v_hbm,
        p2v_hbm,
        z_hbm,
        p1tab,
        p2tab,
        widx_all,
        p1_all,
        p2_all,
        *bufs_and_sems,
    ):
        buf_a = bufs_and_sems[0:3]
        buf_b = bufs_and_sems[3:6]
        gsem_a, gsem_b, wsem_a, wsem_b = bufs_and_sems[6:10]
        wid = lax.axis_index("s") * _NUM_CORES + lax.axis_index("c")
        blk0 = wid * per_w
        tok0 = blk0 * _BLK

        pltpu.sync_copy(p1v_hbm, p1tab)
        pltpu.sync_copy(p2v_hbm, p2tab)
        pltpu.sync_copy(wflat_hbm.at[pl.ds(tok0, per_w * _BLK)], widx_all)
        pltpu.sync_copy(p1flat_hbm.at[pl.ds(tok0, per_w * _BLK)], p1_all)
        pltpu.sync_copy(p2flat_hbm.at[pl.ds(tok0, per_w * _BLK)], p2_all)

        iota = lax.iota(jnp.int32, _LANES)
        tvecs = [iota + g * _LANES for g in range(_BLK // _LANES)]

        def gather_src(blk):
            return wt_hbm.at[
                widx_all.at[pl.ds((blk - blk0) * _BLK, _BLK)]
            ]

        def issue(blk, buf, gsem):
            rows, _, _ = buf
            pltpu.async_copy(gather_src(blk), rows, gsem)

        def z_slices(blk):
            li = blk // bt_per_l
            bt = blk % bt_per_l
            zw = z_hbm.at[li, pl.ds(0, WORD_DIM), pl.ds(bt * _BLK, _BLK)]
            zp = z_hbm.at[li, pl.ds(WORD_DIM, POS_DIM), pl.ds(bt * _BLK, _BLK)]
            return zw, zp

        def process(blk, buf, gsem, wsem, first=False):
            rows, trows, pblk = buf
            loff = (blk - blk0) * _BLK
            zw, zp = z_slices(blk)
            if not first:
                pltpu.make_async_copy(trows, zw, wsem).wait()
                pltpu.make_async_copy(pblk, zp, wsem).wait()

            @plsc.parallel_loop(0, _BLK // _LANES, unroll=2)
            def _pg(g):
                sl = pl.ds(g * _LANES, _LANES)
                isl = pl.ds(loff + g * _LANES, _LANES)
                b1 = p1_all[isl] * (POS_SIZE + 1)
                b2 = p2_all[isl] * (POS_SIZE + 1)
                for f in range(POS_SIZE):
                    pblk[f, sl] = plsc.load_gather(p1tab, [b1 + f])
                    pblk[POS_SIZE + f, sl] = plsc.load_gather(p2tab, [b2 + f])

            pltpu.make_async_copy(gather_src(blk), rows, gsem).wait()
            @plsc.parallel_loop(0, WORD_DIM, unroll=4)
            def _f(i):
                gf = i >> 4
                d = i & 15
                col = (gf * _LANES) + ((d + iota) & 15)
                for g in range(_BLK // _LANES):
                    v = plsc.load_gather(rows, [tvecs[g], col])
                    plsc.store_scatter(trows, [col, tvecs[g]], v)

            pltpu.async_copy(trows, zw, wsem)
            pltpu.async_copy(pblk, zp, wsem)

        issue(blk0, buf_a, gsem_a)
        issue(blk0 + 1, buf_b, gsem_b)
        process(blk0, buf_a, gsem_a, wsem_a, first=True)
        issue(blk0 + 2, buf_a, gsem_a)
        process(blk0 + 1, buf_b, gsem_b, wsem_b, first=True)
        issue(blk0 + 3, buf_b, gsem_b)

        @pl.loop(1, per_w // 2 - 1)
        def _pair(k):
            s = blk0 + 2 * k
            process(s, buf_a, gsem_a, wsem_a)
            issue(s + 2, buf_a, gsem_a)
            process(s + 1, buf_b, gsem_b, wsem_b)
            issue(s + 3, buf_b, gsem_b)

        process(blk0 + per_w - 2, buf_a, gsem_a, wsem_a)
        process(blk0 + per_w - 1, buf_b, gsem_b, wsem_b)
        zw, zp = z_slices(blk0 + per_w - 2)
        pltpu.make_async_copy(buf_a[1], zw, wsem_a).wait()
        pltpu.make_async_copy(buf_a[2], zp, wsem_a).wait()
        zw, zp = z_slices(blk0 + per_w - 1)
        pltpu.make_async_copy(buf_b[1], zw, wsem_b).wait()
        pltpu.make_async_copy(buf_b[2], zp, wsem_b).wait()

    return emb_kernel(wflat, p1flat, p2flat, word_table, p1v, p2v)


def kernel(word, pos1, pos2, word_table, pos1_table, pos2_table):
    b, l = word.shape
    assert b % _BLK == 0 and (b // _BLK) * l % (2 * _NUM_WORKERS) == 0
    wflat = jnp.transpose(word).reshape(-1).astype(jnp.int32)
    p1flat = jnp.transpose(pos1).reshape(-1).astype(jnp.int32)
    p2flat = jnp.transpose(pos2).reshape(-1).astype(jnp.int32)
    p1pad = jnp.pad(pos1_table, ((0, 0), (0, 1))).reshape(-1)
    p2pad = jnp.pad(pos2_table, ((0, 0), (0, 1))).reshape(-1)
    z = _embed(wflat, p1flat, p2flat, word_table, p1pad, p2pad, b, l)
    return jnp.transpose(z, (2, 0, 1))
